# Initial kernel scaffold; baseline (speedup 1.0000x reference)
import jax
import jax.numpy as jnp
from jax.experimental import pallas as pl


def kernel(input_ids, word_table, pos_table, ln_gamma, ln_beta):
    raise NotImplementedError("write your pallas kernel here")


# speedup vs baseline: 1.3948x; 1.3948x over previous
import functools

import jax
import jax.numpy as jnp
from jax import lax
from jax.experimental import pallas as pl
from jax.experimental.pallas import tpu as pltpu
from jax.experimental.pallas import tpu_sc as plsc

_VOCAB = 100000
_D = 128
_L = 200
_B = 1024
_N = _B * _L
_NC = 2
_NS = 16
_NW = _NC * _NS
_TPW = _N // _NW
_C = 128
_NCHUNK = _TPW // _C
_EPS = 1e-12


def _allreduce_sum(v, lanes):
    for k in (1, 2, 4, 8):
        v = v + v.at[lanes ^ k].get(mode="promise_in_bounds")
    return v


def _ln_chunk(rows_v, pos_v, gb_v, p0, n_tok):
    lanes = lax.iota(jnp.int32, 16)

    def tok(i, carry):
        p = p0 + i
        p = p - jnp.where(p >= _L, _L, 0)
        sacc = jnp.zeros((16,), jnp.float32)
        ssacc = jnp.zeros((16,), jnp.float32)
        vs = []
        for j in range(8):
            v = rows_v[i, pl.ds(j * 16, 16)] + pos_v[p, pl.ds(j * 16, 16)]
            vs.append(v)
            sacc = sacc + v
            ssacc = ssacc + v * v
        meanv = _allreduce_sum(sacc, lanes) * (1.0 / _D)
        varv = _allreduce_sum(ssacc, lanes) * (1.0 / _D) - meanv * meanv + _EPS
        bits = plsc.bitcast(varv, jnp.int32)
        y = plsc.bitcast(jnp.int32(0x5F3759DF) - (bits >> 1), jnp.float32)
        for _ in range(3):
            y = y * (1.5 - 0.5 * varv * y * y)
        for j in range(8):
            g = gb_v[0, pl.ds(j * 16, 16)]
            b = gb_v[1, pl.ds(j * 16, 16)]
            rows_v[i, pl.ds(j * 16, 16)] = (vs[j] - meanv) * y * g + b
        return carry

    lax.fori_loop(0, n_tok, tok, 0)


def _sc_embed(ids_hbm, word_hbm, pos_hbm, gamma_hbm, beta_hbm, out_hbm,
              idx_v, pos_v, gb_v, rows_v, sem):
    cid = lax.axis_index("c")
    sid = lax.axis_index("s")
    wid = sid * _NC + cid
    base = wid * _TPW
    pltpu.sync_copy(ids_hbm.at[pl.ds(base, _TPW)], idx_v)
    pltpu.sync_copy(pos_hbm.at[pl.ds(0, _L), :], pos_v)
    pltpu.sync_copy(gamma_hbm, gb_v.at[0])
    pltpu.sync_copy(beta_hbm, gb_v.at[1])

    def chunk(c, carry):
        off = c * _C
        pltpu.async_copy(word_hbm.at[idx_v.at[pl.ds(off, _C)]], rows_v, sem).wait()
        p0 = lax.rem(base + off, _L)
        _ln_chunk(rows_v, pos_v, gb_v, p0, _C)
        pltpu.sync_copy(rows_v, out_hbm.at[pl.ds(base + off, _C), :])
        return carry

    lax.fori_loop(0, _NCHUNK, chunk, 0)


@jax.jit
def _run(ids_flat, word_table, pos_table, ln_gamma, ln_beta):
    mesh = plsc.VectorSubcoreMesh(core_axis_name="c", subcore_axis_name="s")
    call = functools.partial(
        pl.kernel,
        mesh=mesh,
        compiler_params=pltpu.CompilerParams(needs_layout_passes=False),
        out_type=jax.ShapeDtypeStruct((_N, _D), jnp.float32),
        scratch_types=[
            pltpu.VMEM((_TPW,), jnp.int32),
            pltpu.VMEM((_L, _D), jnp.float32),
            pltpu.VMEM((2, _D), jnp.float32),
            pltpu.VMEM((_C, _D), jnp.float32),
            pltpu.SemaphoreType.DMA,
        ],
    )(_sc_embed)
    return call(ids_flat, word_table, pos_table, ln_gamma, ln_beta)


def kernel(input_ids, word_table, pos_table, ln_gamma, ln_beta):
    ids_flat = input_ids.reshape(-1).astype(jnp.int32)
    out = _run(ids_flat, word_table, pos_table, ln_gamma, ln_beta)
    return out.reshape(_B, _L, _D)
